# baseline (speedup 1.0000x reference)
import functools
import jax
import jax.numpy as jnp
from jax import lax
from jax.experimental import pallas as pl
from jax.experimental.pallas import tpu as pltpu
from jax.experimental.pallas import tpu_sc as plsc

VOCAB = 1000
NC, NS = 2, 16
NW = NC * NS
CHUNK = 64


def _body(n_chunks, idx_hbm, table_hbm, out_hbm, idx_v, buf, sem):
    wid = lax.axis_index("s") * NC + lax.axis_index("c")
    pltpu.sync_copy(idx_hbm.at[wid], idx_v)
    base = wid * (n_chunks * CHUNK)

    def chunk(j, carry):
        pltpu.async_copy(table_hbm.at[idx_v.at[j]], buf, sem).wait()
        pltpu.sync_copy(buf, out_hbm.at[pl.ds(base + j * CHUNK, CHUNK)])
        return carry

    lax.fori_loop(0, n_chunks, chunk, 0)


def kernel(context, table):
    b, t = context.shape
    n = b * t
    assert n % (NW * CHUNK) == 0
    n_chunks = n // (NW * CHUNK)
    idx = context.reshape(NW, n_chunks, CHUNK).astype(jnp.int32)

    mesh = plsc.VectorSubcoreMesh(core_axis_name="c", subcore_axis_name="s")
    run = pl.kernel(
        functools.partial(_body, n_chunks),
        out_type=jax.ShapeDtypeStruct((n, VOCAB), jnp.float32),
        mesh=mesh,
        scratch_types=[
            pltpu.VMEM((n_chunks, CHUNK), jnp.int32),
            pltpu.VMEM((CHUNK, VOCAB), jnp.float32),
            pltpu.SemaphoreType.DMA,
        ],
        compiler_params=pltpu.CompilerParams(use_tc_tiling_on_sc=False),
    )
    out = run(idx, table)
    return out.reshape(b, t, VOCAB)


# speedup vs baseline: 1.0112x; 1.0112x over previous
import functools
import jax
import jax.numpy as jnp
from jax import lax
from jax.experimental import pallas as pl
from jax.experimental.pallas import tpu as pltpu
from jax.experimental.pallas import tpu_sc as plsc

VOCAB = 1000
NC, NS = 2, 16
NW = NC * NS
CHUNK = 40


def _body(n_chunks, idx_hbm, table_hbm, out_hbm,
          idx_v, buf0, buf1, gs0, gs1, ss0, ss1):
    wid = lax.axis_index("s") * NC + lax.axis_index("c")
    pltpu.sync_copy(idx_hbm.at[wid], idx_v)
    base = wid * (n_chunks * CHUNK)

    def g_start(c, buf, sem):
        pltpu.async_copy(table_hbm.at[idx_v.at[c]], buf, sem)

    def g_wait(c, buf, sem):
        pltpu.make_async_copy(table_hbm.at[idx_v.at[c]], buf, sem).wait()

    def out_slice(c):
        return out_hbm.at[pl.ds(base + c * CHUNK, CHUNK)]

    def s_start(c, buf, sem):
        pltpu.async_copy(buf, out_slice(c), sem)

    def s_wait(c, buf, sem):
        pltpu.make_async_copy(buf, out_slice(c), sem).wait()

    g_start(0, buf0, gs0)
    g_start(1, buf1, gs1)

    def it(i, carry):
        c0 = 2 * i
        c1 = c0 + 1
        g_wait(c0, buf0, gs0)
        s_start(c0, buf0, ss0)
        g_wait(c1, buf1, gs1)
        s_start(c1, buf1, ss1)
        s_wait(c0, buf0, ss0)
        g_start(c0 + 2, buf0, gs0)
        s_wait(c1, buf1, ss1)
        g_start(c1 + 2, buf1, gs1)
        return carry

    lax.fori_loop(0, n_chunks // 2 - 1, it, 0)

    c0 = n_chunks - 2
    c1 = n_chunks - 1
    g_wait(c0, buf0, gs0)
    s_start(c0, buf0, ss0)
    g_wait(c1, buf1, gs1)
    s_start(c1, buf1, ss1)
    s_wait(c0, buf0, ss0)
    s_wait(c1, buf1, ss1)


def kernel(context, table):
    b, t = context.shape
    n = b * t
    assert n % (NW * CHUNK) == 0
    n_chunks = n // (NW * CHUNK)
    assert n_chunks % 2 == 0
    idx = context.reshape(NW, n_chunks, CHUNK).astype(jnp.int32)

    mesh = plsc.VectorSubcoreMesh(core_axis_name="c", subcore_axis_name="s")
    run = pl.kernel(
        functools.partial(_body, n_chunks),
        out_type=jax.ShapeDtypeStruct((n, VOCAB), jnp.float32),
        mesh=mesh,
        scratch_types=[
            pltpu.VMEM((n_chunks, CHUNK), jnp.int32),
            pltpu.VMEM((CHUNK, VOCAB), jnp.float32),
            pltpu.VMEM((CHUNK, VOCAB), jnp.float32),
            pltpu.SemaphoreType.DMA,
            pltpu.SemaphoreType.DMA,
            pltpu.SemaphoreType.DMA,
            pltpu.SemaphoreType.DMA,
        ],
        compiler_params=pltpu.CompilerParams(use_tc_tiling_on_sc=False),
    )
    out = run(idx, table)
    return out.reshape(b, t, VOCAB)
